# Initial kernel scaffold; baseline (speedup 1.0000x reference)
import jax
import jax.numpy as jnp
from jax.experimental import pallas as pl


def kernel(paper_id, year_idx, aw_edge_index, pv_src, pv_dst, paper_emb, author_emb, venue_emb, w_author, w_venue):
    raise NotImplementedError("write your pallas kernel here")


# speedup vs baseline: 4.7738x; 4.7738x over previous
import jax
import jax.numpy as jnp
from jax import lax
from jax.experimental import pallas as pl
from jax.experimental.pallas import tpu as pltpu
from jax.experimental.pallas import tpu_sc as plsc

D = 256
L = 16
NW = 16
E_AW = 160000
E_PV = 10000
CH_A = E_AW // NW
E_PV_PAD = 10240
CH_V = E_PV_PAD // NW
ACC_ROWS = 2


def _partials_body(pid_hbm, srcA_hbm, dstA_hbm, srcV_hbm, dstV_hbm,
                   author_hbm, venue_hbm, part_hbm, cnt_hbm,
                   pid_v, srcA, dstA, mlA, srcV, dstV, mlV,
                   rows, gidx_v, cell, pacc, cnt_v, sem):
    wid = lax.axis_index("s")
    fzeros = jnp.zeros((L,), jnp.float32)
    zidx = jnp.zeros((L,), jnp.int32)

    pltpu.sync_copy(pid_hbm, pid_v)
    pltpu.sync_copy(srcA_hbm.at[pl.ds(wid * CH_A, CH_A)], srcA)
    pltpu.sync_copy(dstA_hbm.at[pl.ds(wid * CH_A, CH_A)], dstA)
    pltpu.sync_copy(srcV_hbm.at[pl.ds(wid * CH_V, CH_V)], srcV)
    pltpu.sync_copy(dstV_hbm.at[pl.ds(wid * CH_V, CH_V)], dstV)
    pid = pid_v[...]

    for r in range(ACC_ROWS):
        for c in range(D // L):
            pacc[r, pl.ds(c * L, L)] = fzeros

    def scan_phase(cmp_ref, gat_ref, mlist_ref, nvec):
        def it(j, w):
            d = cmp_ref[pl.ds(j * L, L)] - pid
            m = d == 0
            mi = 1 - jnp.minimum(jnp.abs(d), 1)
            cell[...] = zidx
            plsc.addupdate_scatter(cell, [zidx], mi)
            npos = cell[...][0]

            @pl.when(npos > 0)
            def _():
                plsc.store_compressed(
                    mlist_ref.at[pl.ds(w, L)], gat_ref[pl.ds(j * L, L)],
                    mask=m)

            return w + npos

        n = lax.fori_loop(0, nvec, it, jnp.int32(0))
        mlist_ref[pl.ds(n, L)] = jnp.zeros((L,), jnp.int32)
        return n

    n_a = scan_phase(dstA, srcA, mlA, CH_A // L)
    n_v = scan_phase(srcV, dstV, mlV, CH_V // L)

    def gather_phase(mlist_ref, n, table_hbm, acc_row):
        nbat = (n + L - 1) // L

        def bt(b, _):
            gidx_v[...] = mlist_ref[pl.ds(b * L, L)]
            pltpu.async_copy(table_hbm.at[gidx_v], rows, sem).wait()
            nval = jnp.minimum(n - b * L, L)

            def acc_it(j, _):
                for c in range(D // L):
                    pacc[acc_row, pl.ds(c * L, L)] += rows[j, pl.ds(c * L, L)]
                return 0

            lax.fori_loop(0, nval, acc_it, 0)
            return 0

        lax.fori_loop(0, nbat, bt, 0)

    gather_phase(mlA, n_a, author_hbm, 0)
    gather_phase(mlV, n_v, venue_hbm, 1)

    cnt_v[0, pl.ds(0, L)] = jnp.full((L,), n_a, dtype=jnp.int32).astype(
        jnp.float32)
    cnt_v[1, pl.ds(0, L)] = jnp.full((L,), n_v, dtype=jnp.int32).astype(
        jnp.float32)

    pltpu.sync_copy(pacc, part_hbm.at[wid])
    pltpu.sync_copy(cnt_v, cnt_hbm.at[wid])


def _combine_body(wa_hbm, wv_hbm, part_hbm, cnt_hbm, out_hbm,
                  wa_v, wv_v, pslab, clocal, outv):
    wid = lax.axis_index("s")
    fzeros = jnp.zeros((L,), jnp.float32)

    pltpu.sync_copy(wa_hbm, wa_v)
    pltpu.sync_copy(wv_hbm, wv_v)
    pltpu.sync_copy(cnt_hbm, clocal)
    pltpu.sync_copy(part_hbm, pslab)

    ca = fzeros
    cv = fzeros
    sa = fzeros
    sv = fzeros
    for t in range(NW):
        ca = ca + clocal[t, 0, pl.ds(0, L)]
        cv = cv + clocal[t, 1, pl.ds(0, L)]
        sa = sa + pslab[t, 0, pl.ds(wid * L, L)]
        sv = sv + pslab[t, 1, pl.ds(wid * L, L)]

    one = jnp.ones((L,), jnp.float32)
    fa = jnp.where(ca > 0, wa_v[...] / jnp.maximum(ca, one), fzeros)
    fv = jnp.where(cv > 0, wv_v[...] / jnp.maximum(cv, one), fzeros)
    outv[...] = sa * fa + sv * fv
    pltpu.sync_copy(outv, out_hbm.at[pl.ds(wid * L, L)])


def _sc_call(pid16, wa16, wv16, srcA, dstA, srcV, dstV, author_emb, venue_emb):
    mesh = plsc.VectorSubcoreMesh(core_axis_name="c", subcore_axis_name="s",
                                  num_cores=1, num_subcores=NW)
    cparams = pltpu.CompilerParams(needs_layout_passes=False)
    part, cnt = pl.kernel(
        _partials_body,
        out_type=[
            jax.ShapeDtypeStruct((NW, ACC_ROWS, D), jnp.float32),
            jax.ShapeDtypeStruct((NW, 2, L), jnp.float32),
        ],
        mesh=mesh,
        compiler_params=cparams,
        scratch_types=[
            pltpu.VMEM((L,), jnp.int32),
            pltpu.VMEM((CH_A,), jnp.int32),
            pltpu.VMEM((CH_A,), jnp.int32),
            pltpu.VMEM((CH_A + L,), jnp.int32),
            pltpu.VMEM((CH_V,), jnp.int32),
            pltpu.VMEM((CH_V,), jnp.int32),
            pltpu.VMEM((CH_V + L,), jnp.int32),
            pltpu.VMEM((L, D), jnp.float32),
            pltpu.VMEM((L,), jnp.int32),
            pltpu.VMEM((L,), jnp.int32),
            pltpu.VMEM((ACC_ROWS, D), jnp.float32),
            pltpu.VMEM((2, L), jnp.float32),
            pltpu.SemaphoreType.DMA,
        ],
    )(pid16, srcA, dstA, srcV, dstV, author_emb, venue_emb)

    return pl.kernel(
        _combine_body,
        out_type=jax.ShapeDtypeStruct((D,), jnp.float32),
        mesh=mesh,
        compiler_params=cparams,
        scratch_types=[
            pltpu.VMEM((L,), jnp.float32),
            pltpu.VMEM((L,), jnp.float32),
            pltpu.VMEM((NW, ACC_ROWS, D), jnp.float32),
            pltpu.VMEM((NW, 2, L), jnp.float32),
            pltpu.VMEM((L,), jnp.float32),
        ],
    )(wa16, wv16, part, cnt)


def kernel(paper_id, year_idx, aw_edge_index, pv_src, pv_dst, paper_emb,
           author_emb, venue_emb, w_author, w_venue):
    del year_idx, paper_emb
    pid16 = jnp.full((L,), paper_id, dtype=jnp.int32)
    wa16 = jnp.full((L,), w_author, dtype=jnp.float32)
    wv16 = jnp.full((L,), w_venue, dtype=jnp.float32)
    srcA = aw_edge_index[0].astype(jnp.int32)
    dstA = aw_edge_index[1].astype(jnp.int32)
    pad = E_PV_PAD - E_PV
    srcV = jnp.concatenate(
        [pv_src.astype(jnp.int32), jnp.full((pad,), -1, jnp.int32)])
    dstV = jnp.concatenate(
        [pv_dst.astype(jnp.int32), jnp.zeros((pad,), jnp.int32)])
    return _sc_call(pid16, wa16, wv16, srcA, dstA, srcV, dstV,
                    author_emb, venue_emb)


# speedup vs baseline: 7.1368x; 1.4950x over previous
import jax
import jax.numpy as jnp
from jax import lax
from jax.experimental import pallas as pl
from jax.experimental.pallas import tpu as pltpu
from jax.experimental.pallas import tpu_sc as plsc

D = 256
L = 16
NW = 16
E_AW = 160000
E_PV = 10000
CH_A = E_AW // NW
E_PV_PAD = 10240
CH_V = E_PV_PAD // NW
ACC_ROWS = 2


UNROLL = 4


def _partials_body(pid_hbm, aw_hbm, srcV_hbm, dstV_hbm,
                   author_hbm, venue_hbm, part_hbm, cnt_hbm,
                   pid_v, srcA, dstA, mlA, srcV, dstV, mlV,
                   rows, gidx_v, cell, pacc, cnt_v, sem):
    wid = lax.axis_index("s")
    fzeros = jnp.zeros((L,), jnp.float32)
    zidx = jnp.zeros((L,), jnp.int32)

    pltpu.sync_copy(pid_hbm, pid_v)
    pltpu.sync_copy(aw_hbm.at[pl.ds(wid * CH_A, CH_A)], srcA)
    pltpu.sync_copy(aw_hbm.at[pl.ds(E_AW + wid * CH_A, CH_A)], dstA)
    pltpu.sync_copy(srcV_hbm.at[pl.ds(wid * CH_V, CH_V)], srcV)
    pltpu.sync_copy(dstV_hbm.at[pl.ds(wid * CH_V, CH_V)], dstV)
    pid = pid_v[...]

    for r in range(ACC_ROWS):
        for c in range(D // L):
            pacc[r, pl.ds(c * L, L)] = fzeros

    def scan_phase(cmp_ref, gat_ref, mlist_ref, nvec):
        def one_vec(base, w):
            d = cmp_ref[pl.ds(base, L)] - pid
            m = d == 0
            mi = 1 - jnp.minimum(jnp.abs(d), 1)
            cell[...] = zidx
            plsc.addupdate_scatter(cell, [zidx], mi)
            npos = cell[...][0]

            @pl.when(npos > 0)
            def _():
                plsc.store_compressed(
                    mlist_ref.at[pl.ds(w, L)], gat_ref[pl.ds(base, L)],
                    mask=m)

            return w + npos

        def group(g, w):
            base = g * (UNROLL * L)
            anyv = cmp_ref[pl.ds(base, L)] == pid
            for k in range(1, UNROLL):
                anyv = jnp.logical_or(anyv,
                                      cmp_ref[pl.ds(base + k * L, L)] == pid)
            has = jnp.any(anyv)

            def hit(w):
                for k in range(UNROLL):
                    w = one_vec(base + k * L, w)
                return w

            return lax.cond(has, hit, lambda w: w, w)

        ngrp = nvec // UNROLL
        n = lax.fori_loop(0, ngrp, group, jnp.int32(0))
        for k in range(nvec % UNROLL):
            n = one_vec((ngrp * UNROLL + k) * L, n)
        mlist_ref[pl.ds(n, L)] = jnp.zeros((L,), jnp.int32)
        return n

    n_a = scan_phase(dstA, srcA, mlA, CH_A // L)
    n_v = scan_phase(srcV, dstV, mlV, CH_V // L)

    def gather_phase(mlist_ref, n, table_hbm, acc_row):
        nbat = (n + L - 1) // L

        def bt(b, _):
            gidx_v[...] = mlist_ref[pl.ds(b * L, L)]
            pltpu.async_copy(table_hbm.at[gidx_v], rows, sem).wait()
            nval = jnp.minimum(n - b * L, L)

            def acc_it(j, _):
                for c in range(D // L):
                    pacc[acc_row, pl.ds(c * L, L)] += rows[j, pl.ds(c * L, L)]
                return 0

            lax.fori_loop(0, nval, acc_it, 0)
            return 0

        lax.fori_loop(0, nbat, bt, 0)

    gather_phase(mlA, n_a, author_hbm, 0)
    gather_phase(mlV, n_v, venue_hbm, 1)

    cnt_v[0, pl.ds(0, L)] = jnp.full((L,), n_a, dtype=jnp.int32).astype(
        jnp.float32)
    cnt_v[1, pl.ds(0, L)] = jnp.full((L,), n_v, dtype=jnp.int32).astype(
        jnp.float32)

    pltpu.sync_copy(pacc, part_hbm.at[wid])
    pltpu.sync_copy(cnt_v, cnt_hbm.at[wid])


def _combine_body(wa_hbm, wv_hbm, part_hbm, cnt_hbm, out_hbm,
                  wa_v, wv_v, pslab, clocal, outv):
    wid = lax.axis_index("s")
    fzeros = jnp.zeros((L,), jnp.float32)

    pltpu.sync_copy(wa_hbm, wa_v)
    pltpu.sync_copy(wv_hbm, wv_v)
    pltpu.sync_copy(cnt_hbm, clocal)
    pltpu.sync_copy(part_hbm, pslab)

    ca = fzeros
    cv = fzeros
    sa = fzeros
    sv = fzeros
    for t in range(NW):
        ca = ca + clocal[t, 0, pl.ds(0, L)]
        cv = cv + clocal[t, 1, pl.ds(0, L)]
        sa = sa + pslab[t, 0, pl.ds(wid * L, L)]
        sv = sv + pslab[t, 1, pl.ds(wid * L, L)]

    one = jnp.ones((L,), jnp.float32)
    fa = jnp.where(ca > 0, wa_v[...] / jnp.maximum(ca, one), fzeros)
    fv = jnp.where(cv > 0, wv_v[...] / jnp.maximum(cv, one), fzeros)
    outv[...] = sa * fa + sv * fv
    pltpu.sync_copy(outv, out_hbm.at[pl.ds(wid * L, L)])


def _sc_call(pid16, wa16, wv16, aw, srcV, dstV, author_emb, venue_emb):
    mesh = plsc.VectorSubcoreMesh(core_axis_name="c", subcore_axis_name="s",
                                  num_cores=1, num_subcores=NW)
    cparams = pltpu.CompilerParams(needs_layout_passes=False)
    part, cnt = pl.kernel(
        _partials_body,
        out_type=[
            jax.ShapeDtypeStruct((NW, ACC_ROWS, D), jnp.float32),
            jax.ShapeDtypeStruct((NW, 2, L), jnp.float32),
        ],
        mesh=mesh,
        compiler_params=cparams,
        scratch_types=[
            pltpu.VMEM((L,), jnp.int32),
            pltpu.VMEM((CH_A,), jnp.int32),
            pltpu.VMEM((CH_A,), jnp.int32),
            pltpu.VMEM((CH_A + L,), jnp.int32),
            pltpu.VMEM((CH_V,), jnp.int32),
            pltpu.VMEM((CH_V,), jnp.int32),
            pltpu.VMEM((CH_V + L,), jnp.int32),
            pltpu.VMEM((L, D), jnp.float32),
            pltpu.VMEM((L,), jnp.int32),
            pltpu.VMEM((L,), jnp.int32),
            pltpu.VMEM((ACC_ROWS, D), jnp.float32),
            pltpu.VMEM((2, L), jnp.float32),
            pltpu.SemaphoreType.DMA,
        ],
    )(pid16, aw, srcV, dstV, author_emb, venue_emb)

    return pl.kernel(
        _combine_body,
        out_type=jax.ShapeDtypeStruct((D,), jnp.float32),
        mesh=mesh,
        compiler_params=cparams,
        scratch_types=[
            pltpu.VMEM((L,), jnp.float32),
            pltpu.VMEM((L,), jnp.float32),
            pltpu.VMEM((NW, ACC_ROWS, D), jnp.float32),
            pltpu.VMEM((NW, 2, L), jnp.float32),
            pltpu.VMEM((L,), jnp.float32),
        ],
    )(wa16, wv16, part, cnt)


def kernel(paper_id, year_idx, aw_edge_index, pv_src, pv_dst, paper_emb,
           author_emb, venue_emb, w_author, w_venue):
    del year_idx, paper_emb
    pid16 = jnp.full((L,), paper_id, dtype=jnp.int32)
    wa16 = jnp.full((L,), w_author, dtype=jnp.float32)
    wv16 = jnp.full((L,), w_venue, dtype=jnp.float32)
    aw = aw_edge_index.astype(jnp.int32).reshape(-1)
    pad = E_PV_PAD - E_PV
    srcV = jnp.concatenate(
        [pv_src.astype(jnp.int32), jnp.full((pad,), -1, jnp.int32)])
    dstV = jnp.concatenate(
        [pv_dst.astype(jnp.int32), jnp.zeros((pad,), jnp.int32)])
    return _sc_call(pid16, wa16, wv16, aw, srcV, dstV,
                    author_emb, venue_emb)


# speedup vs baseline: 8.3150x; 1.1651x over previous
import jax
import jax.numpy as jnp
from jax import lax
from jax.experimental import pallas as pl
from jax.experimental.pallas import tpu as pltpu
from jax.experimental.pallas import tpu_sc as plsc

D = 256
L = 16
NW = 16
E_AW = 160000
E_PV = 10000
CH_A = E_AW // NW
E_PV_PAD = 10240
CH_V = E_PV_PAD // NW
ACC_ROWS = 2
UNROLL = 4


def _partials_body(pid_hbm, aw_hbm, srcV_hbm, pvd_hbm,
                   author_hbm, venue_hbm, part_hbm, cnt_hbm,
                   pid_v, dstA, srcV, mlA, mlV,
                   rows, gidx_v, sidx_v, cell, pacc, cnt_v, sem):
    wid = lax.axis_index("s")
    fzeros = jnp.zeros((L,), jnp.float32)
    zidx = jnp.zeros((L,), jnp.int32)
    iota = lax.iota(jnp.int32, L)

    pltpu.sync_copy(pid_hbm, pid_v)
    pltpu.sync_copy(aw_hbm.at[pl.ds(E_AW + wid * CH_A, CH_A)], dstA)
    pltpu.sync_copy(srcV_hbm.at[pl.ds(wid * CH_V, CH_V)], srcV)
    pid = pid_v[...]

    for r in range(ACC_ROWS):
        for c in range(D // L):
            pacc[r, pl.ds(c * L, L)] = fzeros

    def scan_phase(cmp_ref, mlist_ref, nvec):
        def one_vec(base, w):
            d = cmp_ref[pl.ds(base, L)] - pid
            m = d == 0
            mi = 1 - jnp.minimum(jnp.abs(d), 1)
            cell[...] = zidx
            plsc.addupdate_scatter(cell, [zidx], mi)
            npos = cell[...][0]

            @pl.when(npos > 0)
            def _():
                plsc.store_compressed(
                    mlist_ref.at[pl.ds(w, L)], base + iota, mask=m)

            return w + npos

        def group(g, w):
            base = g * (UNROLL * L)
            anyv = cmp_ref[pl.ds(base, L)] == pid
            for k in range(1, UNROLL):
                anyv = jnp.logical_or(anyv,
                                      cmp_ref[pl.ds(base + k * L, L)] == pid)
            has = jnp.any(anyv)

            def hit(w):
                for k in range(UNROLL):
                    w = one_vec(base + k * L, w)
                return w

            return lax.cond(has, hit, lambda w: w, w)

        ngrp = nvec // UNROLL
        n = lax.fori_loop(0, ngrp, group, jnp.int32(0))
        for k in range(nvec % UNROLL):
            n = one_vec((ngrp * UNROLL + k) * L, n)
        mlist_ref[pl.ds(n, L)] = jnp.zeros((L,), jnp.int32)
        return n

    n_a = scan_phase(dstA, mlA, CH_A // L)
    n_v = scan_phase(srcV, mlV, CH_V // L)

    def gather_phase(mlist_ref, n, pos_base, id_hbm, table_hbm, acc_row):
        nbat = (n + L - 1) // L

        def bt(b, _):
            gidx_v[...] = mlist_ref[pl.ds(b * L, L)] + pos_base
            pltpu.async_copy(id_hbm.at[gidx_v], sidx_v, sem).wait()
            pltpu.async_copy(table_hbm.at[sidx_v], rows, sem).wait()
            nval = jnp.minimum(n - b * L, L)

            def acc_it(j, _):
                for c in range(D // L):
                    pacc[acc_row, pl.ds(c * L, L)] += rows[j, pl.ds(c * L, L)]
                return 0

            lax.fori_loop(0, nval, acc_it, 0)
            return 0

        lax.fori_loop(0, nbat, bt, 0)

    gather_phase(mlA, n_a, wid * CH_A, aw_hbm, author_hbm, 0)
    gather_phase(mlV, n_v, wid * CH_V, pvd_hbm, venue_hbm, 1)

    cnt_v[0, pl.ds(0, L)] = jnp.full((L,), n_a, dtype=jnp.int32).astype(
        jnp.float32)
    cnt_v[1, pl.ds(0, L)] = jnp.full((L,), n_v, dtype=jnp.int32).astype(
        jnp.float32)

    pltpu.sync_copy(pacc, part_hbm.at[wid])
    pltpu.sync_copy(cnt_v, cnt_hbm.at[wid])


def _combine_body(wa_hbm, wv_hbm, part_hbm, cnt_hbm, out_hbm,
                  wa_v, wv_v, pslab, clocal, outv):
    wid = lax.axis_index("s")
    fzeros = jnp.zeros((L,), jnp.float32)

    pltpu.sync_copy(wa_hbm, wa_v)
    pltpu.sync_copy(wv_hbm, wv_v)
    pltpu.sync_copy(cnt_hbm, clocal)
    pltpu.sync_copy(part_hbm, pslab)

    ca = fzeros
    cv = fzeros
    sa = fzeros
    sv = fzeros
    for t in range(NW):
        ca = ca + clocal[t, 0, pl.ds(0, L)]
        cv = cv + clocal[t, 1, pl.ds(0, L)]
        sa = sa + pslab[t, 0, pl.ds(wid * L, L)]
        sv = sv + pslab[t, 1, pl.ds(wid * L, L)]

    one = jnp.ones((L,), jnp.float32)
    fa = jnp.where(ca > 0, wa_v[...] / jnp.maximum(ca, one), fzeros)
    fv = jnp.where(cv > 0, wv_v[...] / jnp.maximum(cv, one), fzeros)
    outv[...] = sa * fa + sv * fv
    pltpu.sync_copy(outv, out_hbm.at[pl.ds(wid * L, L)])


def _sc_call(pid16, wa16, wv16, aw, srcV, pvd, author_emb, venue_emb):
    mesh = plsc.VectorSubcoreMesh(core_axis_name="c", subcore_axis_name="s",
                                  num_cores=1, num_subcores=NW)
    cparams = pltpu.CompilerParams(needs_layout_passes=False)
    part, cnt = pl.kernel(
        _partials_body,
        out_type=[
            jax.ShapeDtypeStruct((NW, ACC_ROWS, D), jnp.float32),
            jax.ShapeDtypeStruct((NW, 2, L), jnp.float32),
        ],
        mesh=mesh,
        compiler_params=cparams,
        scratch_types=[
            pltpu.VMEM((L,), jnp.int32),
            pltpu.VMEM((CH_A,), jnp.int32),
            pltpu.VMEM((CH_V,), jnp.int32),
            pltpu.VMEM((CH_A + L,), jnp.int32),
            pltpu.VMEM((CH_V + L,), jnp.int32),
            pltpu.VMEM((L, D), jnp.float32),
            pltpu.VMEM((L,), jnp.int32),
            pltpu.VMEM((L,), jnp.int32),
            pltpu.VMEM((L,), jnp.int32),
            pltpu.VMEM((ACC_ROWS, D), jnp.float32),
            pltpu.VMEM((2, L), jnp.float32),
            pltpu.SemaphoreType.DMA,
        ],
    )(pid16, aw, srcV, pvd, author_emb, venue_emb)

    return pl.kernel(
        _combine_body,
        out_type=jax.ShapeDtypeStruct((D,), jnp.float32),
        mesh=mesh,
        compiler_params=cparams,
        scratch_types=[
            pltpu.VMEM((L,), jnp.float32),
            pltpu.VMEM((L,), jnp.float32),
            pltpu.VMEM((NW, ACC_ROWS, D), jnp.float32),
            pltpu.VMEM((NW, 2, L), jnp.float32),
            pltpu.VMEM((L,), jnp.float32),
        ],
    )(wa16, wv16, part, cnt)


def kernel(paper_id, year_idx, aw_edge_index, pv_src, pv_dst, paper_emb,
           author_emb, venue_emb, w_author, w_venue):
    del year_idx, paper_emb
    pid16 = jnp.full((L,), paper_id, dtype=jnp.int32)
    wa16 = jnp.full((L,), w_author, dtype=jnp.float32)
    wv16 = jnp.full((L,), w_venue, dtype=jnp.float32)
    aw = aw_edge_index.astype(jnp.int32).reshape(-1)
    pad = E_PV_PAD - E_PV
    srcV = jnp.concatenate(
        [pv_src.astype(jnp.int32), jnp.full((pad,), -1, jnp.int32)])
    return _sc_call(pid16, wa16, wv16, aw, srcV, pv_dst.astype(jnp.int32),
                    author_emb, venue_emb)


# speedup vs baseline: 8.5302x; 1.0259x over previous
import jax
import jax.numpy as jnp
from jax import lax
from jax.experimental import pallas as pl
from jax.experimental.pallas import tpu as pltpu
from jax.experimental.pallas import tpu_sc as plsc

D = 256
L = 16
NW = 16
E_AW = 160000
E_PV = 10000
CH_A = E_AW // NW
E_PV_PAD = 10240
CH_V = E_PV_PAD // NW
ACC_ROWS = 2
UNROLL = 4


def _fused_body(pid_hbm, wa_hbm, wv_hbm, aw_hbm, srcV_hbm, pvd_hbm,
                author_hbm, venue_hbm,
                out_hbm, part_hbm, cnt_hbm,
                pid_v, dstA, srcV, mlA, mlV,
                rows, gidx_v, sidx_v, cell, pacc, cnt_v,
                wa_v, wv_v, pslab, clocal, outv, sem):
    wid = lax.axis_index("s")
    fzeros = jnp.zeros((L,), jnp.float32)
    zidx = jnp.zeros((L,), jnp.int32)
    iota = lax.iota(jnp.int32, L)

    pltpu.sync_copy(pid_hbm, pid_v)
    pltpu.sync_copy(aw_hbm.at[pl.ds(E_AW + wid * CH_A, CH_A)], dstA)
    pltpu.sync_copy(srcV_hbm.at[pl.ds(wid * CH_V, CH_V)], srcV)
    pltpu.sync_copy(wa_hbm, wa_v)
    pltpu.sync_copy(wv_hbm, wv_v)
    pid = pid_v[...]

    for r in range(ACC_ROWS):
        for c in range(D // L):
            pacc[r, pl.ds(c * L, L)] = fzeros

    def scan_phase(cmp_ref, mlist_ref, nvec):
        def one_vec(base, w):
            d = cmp_ref[pl.ds(base, L)] - pid
            m = d == 0
            mi = 1 - jnp.minimum(jnp.abs(d), 1)
            cell[...] = zidx
            plsc.addupdate_scatter(cell, [zidx], mi)
            npos = cell[...][0]

            @pl.when(npos > 0)
            def _():
                plsc.store_compressed(
                    mlist_ref.at[pl.ds(w, L)], base + iota, mask=m)

            return w + npos

        def group(g, w):
            base = g * (UNROLL * L)
            anyv = cmp_ref[pl.ds(base, L)] == pid
            for k in range(1, UNROLL):
                anyv = jnp.logical_or(anyv,
                                      cmp_ref[pl.ds(base + k * L, L)] == pid)
            has = jnp.any(anyv)

            def hit(w):
                for k in range(UNROLL):
                    w = one_vec(base + k * L, w)
                return w

            return lax.cond(has, hit, lambda w: w, w)

        ngrp = nvec // UNROLL
        n = lax.fori_loop(0, ngrp, group, jnp.int32(0))
        for k in range(nvec % UNROLL):
            n = one_vec((ngrp * UNROLL + k) * L, n)
        mlist_ref[pl.ds(n, L)] = jnp.zeros((L,), jnp.int32)
        return n

    n_a = scan_phase(dstA, mlA, CH_A // L)
    n_v = scan_phase(srcV, mlV, CH_V // L)

    def gather_phase(mlist_ref, n, pos_base, id_hbm, table_hbm, acc_row):
        nbat = (n + L - 1) // L

        def bt(b, _):
            gidx_v[...] = mlist_ref[pl.ds(b * L, L)] + pos_base
            pltpu.async_copy(id_hbm.at[gidx_v], sidx_v, sem).wait()
            pltpu.async_copy(table_hbm.at[sidx_v], rows, sem).wait()
            nval = jnp.minimum(n - b * L, L)

            def acc_it(j, _):
                for c in range(D // L):
                    pacc[acc_row, pl.ds(c * L, L)] += rows[j, pl.ds(c * L, L)]
                return 0

            lax.fori_loop(0, nval, acc_it, 0)
            return 0

        lax.fori_loop(0, nbat, bt, 0)

    gather_phase(mlA, n_a, wid * CH_A, aw_hbm, author_hbm, 0)
    gather_phase(mlV, n_v, wid * CH_V, pvd_hbm, venue_hbm, 1)

    cnt_v[0, pl.ds(0, L)] = jnp.full((L,), n_a, dtype=jnp.int32).astype(
        jnp.float32)
    cnt_v[1, pl.ds(0, L)] = jnp.full((L,), n_v, dtype=jnp.int32).astype(
        jnp.float32)

    pltpu.sync_copy(pacc, part_hbm.at[wid])
    pltpu.sync_copy(cnt_v, cnt_hbm.at[wid])

    plsc.subcore_barrier()

    pltpu.sync_copy(part_hbm, pslab)
    pltpu.sync_copy(cnt_hbm, clocal)

    ca = fzeros
    cv = fzeros
    sa = fzeros
    sv = fzeros
    for t in range(NW):
        ca = ca + clocal[t, 0, pl.ds(0, L)]
        cv = cv + clocal[t, 1, pl.ds(0, L)]
        sa = sa + pslab[t, 0, pl.ds(wid * L, L)]
        sv = sv + pslab[t, 1, pl.ds(wid * L, L)]

    one = jnp.ones((L,), jnp.float32)
    fa = jnp.where(ca > 0, wa_v[...] / jnp.maximum(ca, one), fzeros)
    fv = jnp.where(cv > 0, wv_v[...] / jnp.maximum(cv, one), fzeros)
    outv[...] = sa * fa + sv * fv
    pltpu.sync_copy(outv, out_hbm.at[pl.ds(wid * L, L)])


def _sc_call(pid16, wa16, wv16, aw, srcV, pvd, author_emb, venue_emb):
    mesh = plsc.VectorSubcoreMesh(core_axis_name="c", subcore_axis_name="s",
                                  num_cores=1, num_subcores=NW)
    cparams = pltpu.CompilerParams(needs_layout_passes=False)
    out, _, _ = pl.kernel(
        _fused_body,
        out_type=[
            jax.ShapeDtypeStruct((D,), jnp.float32),
            jax.ShapeDtypeStruct((NW, ACC_ROWS, D), jnp.float32),
            jax.ShapeDtypeStruct((NW, 2, L), jnp.float32),
        ],
        mesh=mesh,
        compiler_params=cparams,
        scratch_types=[
            pltpu.VMEM((L,), jnp.int32),
            pltpu.VMEM((CH_A,), jnp.int32),
            pltpu.VMEM((CH_V,), jnp.int32),
            pltpu.VMEM((CH_A + L,), jnp.int32),
            pltpu.VMEM((CH_V + L,), jnp.int32),
            pltpu.VMEM((L, D), jnp.float32),
            pltpu.VMEM((L,), jnp.int32),
            pltpu.VMEM((L,), jnp.int32),
            pltpu.VMEM((L,), jnp.int32),
            pltpu.VMEM((ACC_ROWS, D), jnp.float32),
            pltpu.VMEM((2, L), jnp.float32),
            pltpu.VMEM((L,), jnp.float32),
            pltpu.VMEM((L,), jnp.float32),
            pltpu.VMEM((NW, ACC_ROWS, D), jnp.float32),
            pltpu.VMEM((NW, 2, L), jnp.float32),
            pltpu.VMEM((L,), jnp.float32),
            pltpu.SemaphoreType.DMA,
        ],
    )(pid16, wa16, wv16, aw, srcV, pvd, author_emb, venue_emb)
    return out


def kernel(paper_id, year_idx, aw_edge_index, pv_src, pv_dst, paper_emb,
           author_emb, venue_emb, w_author, w_venue):
    del year_idx, paper_emb
    pid16 = jnp.full((L,), paper_id, dtype=jnp.int32)
    wa16 = jnp.full((L,), w_author, dtype=jnp.float32)
    wv16 = jnp.full((L,), w_venue, dtype=jnp.float32)
    aw = aw_edge_index.astype(jnp.int32).reshape(-1)
    pad = E_PV_PAD - E_PV
    srcV = jnp.concatenate(
        [pv_src.astype(jnp.int32), jnp.full((pad,), -1, jnp.int32)])
    return _sc_call(pid16, wa16, wv16, aw, srcV, pv_dst.astype(jnp.int32),
                    author_emb, venue_emb)


# speedup vs baseline: 8.9918x; 1.0541x over previous
import jax
import jax.numpy as jnp
from jax import lax
from jax.experimental import pallas as pl
from jax.experimental.pallas import tpu as pltpu
from jax.experimental.pallas import tpu_sc as plsc

D = 256
L = 16
NW = 16
E_AW = 160000
E_PV = 10000
CH_A = E_AW // NW
E_PV_PAD = 10240
CH_V = E_PV_PAD // NW
ACC_ROWS = 2
UNROLL = 8


def _fused_body(pid_hbm, wa_hbm, wv_hbm, aw_hbm, srcV_hbm, pvd_hbm,
                author_hbm, venue_hbm,
                out_hbm, part_hbm, cnt_hbm,
                pid_v, dstA, srcV, mlA, mlV,
                rows, gidx_v, sidx_v, cell, pacc, cnt_v,
                wa_v, wv_v, pslab, clocal, outv, sem):
    wid = lax.axis_index("s")
    fzeros = jnp.zeros((L,), jnp.float32)
    zidx = jnp.zeros((L,), jnp.int32)
    iota = lax.iota(jnp.int32, L)

    pltpu.sync_copy(pid_hbm, pid_v)
    pltpu.sync_copy(aw_hbm.at[pl.ds(E_AW + wid * CH_A, CH_A)], dstA)
    pltpu.sync_copy(srcV_hbm.at[pl.ds(wid * CH_V, CH_V)], srcV)
    pltpu.sync_copy(wa_hbm, wa_v)
    pltpu.sync_copy(wv_hbm, wv_v)
    pid = pid_v[...]

    for r in range(ACC_ROWS):
        for c in range(D // L):
            pacc[r, pl.ds(c * L, L)] = fzeros

    def scan_phase(cmp_ref, mlist_ref, nvec):
        def one_vec(base, w):
            d = cmp_ref[pl.ds(base, L)] - pid
            m = d == 0
            mi = 1 - jnp.minimum(jnp.abs(d), 1)
            cell[...] = zidx
            plsc.addupdate_scatter(cell, [zidx], mi)
            npos = cell[...][0]

            @pl.when(npos > 0)
            def _():
                plsc.store_compressed(
                    mlist_ref.at[pl.ds(w, L)], base + iota, mask=m)

            return w + npos

        def group(g, w):
            base = g * (UNROLL * L)
            anyv = cmp_ref[pl.ds(base, L)] == pid
            for k in range(1, UNROLL):
                anyv = jnp.logical_or(anyv,
                                      cmp_ref[pl.ds(base + k * L, L)] == pid)
            has = jnp.any(anyv)

            def hit(w):
                for k in range(UNROLL):
                    w = one_vec(base + k * L, w)
                return w

            return lax.cond(has, hit, lambda w: w, w)

        ngrp = nvec // UNROLL
        n = lax.fori_loop(0, ngrp, group, jnp.int32(0))
        for k in range(nvec % UNROLL):
            n = one_vec((ngrp * UNROLL + k) * L, n)
        mlist_ref[pl.ds(n, L)] = jnp.zeros((L,), jnp.int32)
        return n

    n_a = scan_phase(dstA, mlA, CH_A // L)
    n_v = scan_phase(srcV, mlV, CH_V // L)

    def gather_phase(mlist_ref, n, pos_base, id_hbm, table_hbm, acc_row):
        nbat = (n + L - 1) // L

        def bt(b, _):
            gidx_v[...] = mlist_ref[pl.ds(b * L, L)] + pos_base
            pltpu.async_copy(id_hbm.at[gidx_v], sidx_v, sem).wait()
            pltpu.async_copy(table_hbm.at[sidx_v], rows, sem).wait()
            nval = jnp.minimum(n - b * L, L)

            def acc_it(j, _):
                for c in range(D // L):
                    pacc[acc_row, pl.ds(c * L, L)] += rows[j, pl.ds(c * L, L)]
                return 0

            lax.fori_loop(0, nval, acc_it, 0)
            return 0

        lax.fori_loop(0, nbat, bt, 0)

    gather_phase(mlA, n_a, wid * CH_A, aw_hbm, author_hbm, 0)
    gather_phase(mlV, n_v, wid * CH_V, pvd_hbm, venue_hbm, 1)

    cnt_v[0, pl.ds(0, L)] = jnp.full((L,), n_a, dtype=jnp.int32).astype(
        jnp.float32)
    cnt_v[1, pl.ds(0, L)] = jnp.full((L,), n_v, dtype=jnp.int32).astype(
        jnp.float32)

    pltpu.sync_copy(pacc, part_hbm.at[wid])
    pltpu.sync_copy(cnt_v, cnt_hbm.at[wid])

    plsc.subcore_barrier()

    pltpu.sync_copy(part_hbm, pslab)
    pltpu.sync_copy(cnt_hbm, clocal)

    ca = fzeros
    cv = fzeros
    sa = fzeros
    sv = fzeros
    for t in range(NW):
        ca = ca + clocal[t, 0, pl.ds(0, L)]
        cv = cv + clocal[t, 1, pl.ds(0, L)]
        sa = sa + pslab[t, 0, pl.ds(wid * L, L)]
        sv = sv + pslab[t, 1, pl.ds(wid * L, L)]

    one = jnp.ones((L,), jnp.float32)
    fa = jnp.where(ca > 0, wa_v[...] / jnp.maximum(ca, one), fzeros)
    fv = jnp.where(cv > 0, wv_v[...] / jnp.maximum(cv, one), fzeros)
    outv[...] = sa * fa + sv * fv
    pltpu.sync_copy(outv, out_hbm.at[pl.ds(wid * L, L)])


def _sc_call(pid16, wa16, wv16, aw, srcV, pvd, author_emb, venue_emb):
    mesh = plsc.VectorSubcoreMesh(core_axis_name="c", subcore_axis_name="s",
                                  num_cores=1, num_subcores=NW)
    cparams = pltpu.CompilerParams(needs_layout_passes=False)
    out, _, _ = pl.kernel(
        _fused_body,
        out_type=[
            jax.ShapeDtypeStruct((D,), jnp.float32),
            jax.ShapeDtypeStruct((NW, ACC_ROWS, D), jnp.float32),
            jax.ShapeDtypeStruct((NW, 2, L), jnp.float32),
        ],
        mesh=mesh,
        compiler_params=cparams,
        scratch_types=[
            pltpu.VMEM((L,), jnp.int32),
            pltpu.VMEM((CH_A,), jnp.int32),
            pltpu.VMEM((CH_V,), jnp.int32),
            pltpu.VMEM((CH_A + L,), jnp.int32),
            pltpu.VMEM((CH_V + L,), jnp.int32),
            pltpu.VMEM((L, D), jnp.float32),
            pltpu.VMEM((L,), jnp.int32),
            pltpu.VMEM((L,), jnp.int32),
            pltpu.VMEM((L,), jnp.int32),
            pltpu.VMEM((ACC_ROWS, D), jnp.float32),
            pltpu.VMEM((2, L), jnp.float32),
            pltpu.VMEM((L,), jnp.float32),
            pltpu.VMEM((L,), jnp.float32),
            pltpu.VMEM((NW, ACC_ROWS, D), jnp.float32),
            pltpu.VMEM((NW, 2, L), jnp.float32),
            pltpu.VMEM((L,), jnp.float32),
            pltpu.SemaphoreType.DMA,
        ],
    )(pid16, wa16, wv16, aw, srcV, pvd, author_emb, venue_emb)
    return out


def kernel(paper_id, year_idx, aw_edge_index, pv_src, pv_dst, paper_emb,
           author_emb, venue_emb, w_author, w_venue):
    del year_idx, paper_emb
    pid16 = jnp.full((L,), paper_id, dtype=jnp.int32)
    wa16 = jnp.full((L,), w_author, dtype=jnp.float32)
    wv16 = jnp.full((L,), w_venue, dtype=jnp.float32)
    aw = aw_edge_index.astype(jnp.int32).reshape(-1)
    pad = E_PV_PAD - E_PV
    srcV = jnp.concatenate(
        [pv_src.astype(jnp.int32), jnp.full((pad,), -1, jnp.int32)])
    return _sc_call(pid16, wa16, wv16, aw, srcV, pv_dst.astype(jnp.int32),
                    author_emb, venue_emb)


# speedup vs baseline: 9.0017x; 1.0011x over previous
import jax
import jax.numpy as jnp
from jax import lax
from jax.experimental import pallas as pl
from jax.experimental.pallas import tpu as pltpu
from jax.experimental.pallas import tpu_sc as plsc

D = 256
L = 16
NW = 16
E_AW = 160000
E_PV = 10000
CH_A = E_AW // NW
E_PV_PAD = 10240
CH_V = E_PV_PAD // NW
ACC_ROWS = 2
UNROLL = 16


def _fused_body(pid_hbm, wa_hbm, wv_hbm, aw_hbm, srcV_hbm, pvd_hbm,
                author_hbm, venue_hbm,
                out_hbm, part_hbm, cnt_hbm,
                pid_v, dstA, srcV, mlA, mlV,
                rows, gidx_v, sidx_v, cell, pacc, cnt_v,
                wa_v, wv_v, pslab, clocal, outv, sem):
    wid = lax.axis_index("s")
    fzeros = jnp.zeros((L,), jnp.float32)
    zidx = jnp.zeros((L,), jnp.int32)
    iota = lax.iota(jnp.int32, L)

    pltpu.sync_copy(pid_hbm, pid_v)
    pltpu.sync_copy(aw_hbm.at[pl.ds(E_AW + wid * CH_A, CH_A)], dstA)
    pltpu.sync_copy(srcV_hbm.at[pl.ds(wid * CH_V, CH_V)], srcV)
    pltpu.sync_copy(wa_hbm, wa_v)
    pltpu.sync_copy(wv_hbm, wv_v)
    pid = pid_v[...]

    for r in range(ACC_ROWS):
        for c in range(D // L):
            pacc[r, pl.ds(c * L, L)] = fzeros

    def scan_phase(cmp_ref, mlist_ref, nvec):
        def one_vec(base, w):
            d = cmp_ref[pl.ds(base, L)] - pid
            m = d == 0
            mi = 1 - jnp.minimum(jnp.abs(d), 1)
            cell[...] = zidx
            plsc.addupdate_scatter(cell, [zidx], mi)
            npos = cell[...][0]

            @pl.when(npos > 0)
            def _():
                plsc.store_compressed(
                    mlist_ref.at[pl.ds(w, L)], base + iota, mask=m)

            return w + npos

        def group(g, w):
            base = g * (UNROLL * L)
            anyv = cmp_ref[pl.ds(base, L)] == pid
            for k in range(1, UNROLL):
                anyv = jnp.logical_or(anyv,
                                      cmp_ref[pl.ds(base + k * L, L)] == pid)
            has = jnp.any(anyv)

            def hit(w):
                for k in range(UNROLL):
                    w = one_vec(base + k * L, w)
                return w

            return lax.cond(has, hit, lambda w: w, w)

        ngrp = nvec // UNROLL
        n = lax.fori_loop(0, ngrp, group, jnp.int32(0))
        for k in range(nvec % UNROLL):
            n = one_vec((ngrp * UNROLL + k) * L, n)
        mlist_ref[pl.ds(n, L)] = jnp.zeros((L,), jnp.int32)
        return n

    n_a = scan_phase(dstA, mlA, CH_A // L)
    n_v = scan_phase(srcV, mlV, CH_V // L)

    def gather_phase(mlist_ref, n, pos_base, id_hbm, table_hbm, acc_row):
        nbat = (n + L - 1) // L

        def bt(b, _):
            gidx_v[...] = mlist_ref[pl.ds(b * L, L)] + pos_base
            pltpu.async_copy(id_hbm.at[gidx_v], sidx_v, sem).wait()
            pltpu.async_copy(table_hbm.at[sidx_v], rows, sem).wait()
            nval = jnp.minimum(n - b * L, L)

            def acc_it(j, _):
                for c in range(D // L):
                    pacc[acc_row, pl.ds(c * L, L)] += rows[j, pl.ds(c * L, L)]
                return 0

            lax.fori_loop(0, nval, acc_it, 0)
            return 0

        lax.fori_loop(0, nbat, bt, 0)

    gather_phase(mlA, n_a, wid * CH_A, aw_hbm, author_hbm, 0)
    gather_phase(mlV, n_v, wid * CH_V, pvd_hbm, venue_hbm, 1)

    cnt_v[0, pl.ds(0, L)] = jnp.full((L,), n_a, dtype=jnp.int32).astype(
        jnp.float32)
    cnt_v[1, pl.ds(0, L)] = jnp.full((L,), n_v, dtype=jnp.int32).astype(
        jnp.float32)

    pltpu.sync_copy(pacc, part_hbm.at[wid])
    pltpu.sync_copy(cnt_v, cnt_hbm.at[wid])

    plsc.subcore_barrier()

    pltpu.sync_copy(part_hbm, pslab)
    pltpu.sync_copy(cnt_hbm, clocal)

    ca = fzeros
    cv = fzeros
    sa = fzeros
    sv = fzeros
    for t in range(NW):
        ca = ca + clocal[t, 0, pl.ds(0, L)]
        cv = cv + clocal[t, 1, pl.ds(0, L)]
        sa = sa + pslab[t, 0, pl.ds(wid * L, L)]
        sv = sv + pslab[t, 1, pl.ds(wid * L, L)]

    one = jnp.ones((L,), jnp.float32)
    fa = jnp.where(ca > 0, wa_v[...] / jnp.maximum(ca, one), fzeros)
    fv = jnp.where(cv > 0, wv_v[...] / jnp.maximum(cv, one), fzeros)
    outv[...] = sa * fa + sv * fv
    pltpu.sync_copy(outv, out_hbm.at[pl.ds(wid * L, L)])


def _sc_call(pid16, wa16, wv16, aw, srcV, pvd, author_emb, venue_emb):
    mesh = plsc.VectorSubcoreMesh(core_axis_name="c", subcore_axis_name="s",
                                  num_cores=1, num_subcores=NW)
    cparams = pltpu.CompilerParams(needs_layout_passes=False)
    out, _, _ = pl.kernel(
        _fused_body,
        out_type=[
            jax.ShapeDtypeStruct((D,), jnp.float32),
            jax.ShapeDtypeStruct((NW, ACC_ROWS, D), jnp.float32),
            jax.ShapeDtypeStruct((NW, 2, L), jnp.float32),
        ],
        mesh=mesh,
        compiler_params=cparams,
        scratch_types=[
            pltpu.VMEM((L,), jnp.int32),
            pltpu.VMEM((CH_A,), jnp.int32),
            pltpu.VMEM((CH_V,), jnp.int32),
            pltpu.VMEM((CH_A + L,), jnp.int32),
            pltpu.VMEM((CH_V + L,), jnp.int32),
            pltpu.VMEM((L, D), jnp.float32),
            pltpu.VMEM((L,), jnp.int32),
            pltpu.VMEM((L,), jnp.int32),
            pltpu.VMEM((L,), jnp.int32),
            pltpu.VMEM((ACC_ROWS, D), jnp.float32),
            pltpu.VMEM((2, L), jnp.float32),
            pltpu.VMEM((L,), jnp.float32),
            pltpu.VMEM((L,), jnp.float32),
            pltpu.VMEM((NW, ACC_ROWS, D), jnp.float32),
            pltpu.VMEM((NW, 2, L), jnp.float32),
            pltpu.VMEM((L,), jnp.float32),
            pltpu.SemaphoreType.DMA,
        ],
    )(pid16, wa16, wv16, aw, srcV, pvd, author_emb, venue_emb)
    return out


def kernel(paper_id, year_idx, aw_edge_index, pv_src, pv_dst, paper_emb,
           author_emb, venue_emb, w_author, w_venue):
    del year_idx, paper_emb
    pid16 = jnp.full((L,), paper_id, dtype=jnp.int32)
    wa16 = jnp.full((L,), w_author, dtype=jnp.float32)
    wv16 = jnp.full((L,), w_venue, dtype=jnp.float32)
    aw = aw_edge_index.astype(jnp.int32).reshape(-1)
    pad = E_PV_PAD - E_PV
    srcV = jnp.concatenate(
        [pv_src.astype(jnp.int32), jnp.full((pad,), -1, jnp.int32)])
    return _sc_call(pid16, wa16, wv16, aw, srcV, pv_dst.astype(jnp.int32),
                    author_emb, venue_emb)


# speedup vs baseline: 9.2002x; 1.0221x over previous
import jax
import jax.numpy as jnp
from jax import lax
from jax.experimental import pallas as pl
from jax.experimental.pallas import tpu as pltpu
from jax.experimental.pallas import tpu_sc as plsc

D = 256
L = 16
NW = 16
E_AW = 160000
E_PV = 10000
CH_A = E_AW // NW
E_PV_PAD = 10240
CH_V = E_PV_PAD // NW
CH_V_LAST = E_PV - (NW - 1) * CH_V
ACC_ROWS = 2
UNROLL = 16


def _fused_body(pid_hbm, wa_hbm, wv_hbm, aw_hbm, srcV_hbm, pvd_hbm,
                author_hbm, venue_hbm,
                out_hbm, part_hbm, cnt_hbm,
                pid_v, dstA, srcV, mlA, mlV,
                rows, gidx_v, sidx_v, cell, pacc, cnt_v,
                wa_v, wv_v, pslab, clocal, outv, sem):
    wid = lax.axis_index("s")
    fzeros = jnp.zeros((L,), jnp.float32)
    zidx = jnp.zeros((L,), jnp.int32)
    iota = lax.iota(jnp.int32, L)

    pltpu.sync_copy(pid_hbm, pid_v)
    pltpu.sync_copy(aw_hbm.at[pl.ds(E_AW + wid * CH_A, CH_A)], dstA)
    @pl.when(wid < NW - 1)
    def _():
        pltpu.sync_copy(srcV_hbm.at[pl.ds(wid * CH_V, CH_V)], srcV)

    @pl.when(wid == NW - 1)
    def _():
        pltpu.sync_copy(srcV_hbm.at[pl.ds(E_PV - CH_V_LAST, CH_V_LAST)],
                        srcV.at[pl.ds(0, CH_V_LAST)])
        neg1 = jnp.full((L,), -1, jnp.int32)
        for k in range(CH_V_LAST, CH_V, L):
            srcV[pl.ds(k, L)] = neg1

    pltpu.sync_copy(wa_hbm, wa_v)
    pltpu.sync_copy(wv_hbm, wv_v)
    pid = pid_v[...]

    for r in range(ACC_ROWS):
        for c in range(D // L):
            pacc[r, pl.ds(c * L, L)] = fzeros

    def scan_phase(cmp_ref, mlist_ref, nvec):
        def one_vec(base, w):
            d = cmp_ref[pl.ds(base, L)] - pid
            m = d == 0
            mi = 1 - jnp.minimum(jnp.abs(d), 1)
            cell[...] = zidx
            plsc.addupdate_scatter(cell, [zidx], mi)
            npos = cell[...][0]

            @pl.when(npos > 0)
            def _():
                plsc.store_compressed(
                    mlist_ref.at[pl.ds(w, L)], base + iota, mask=m)

            return w + npos

        def group(g, w):
            base = g * (UNROLL * L)
            anyv = cmp_ref[pl.ds(base, L)] == pid
            for k in range(1, UNROLL):
                anyv = jnp.logical_or(anyv,
                                      cmp_ref[pl.ds(base + k * L, L)] == pid)
            has = jnp.any(anyv)

            def hit(w):
                for k in range(UNROLL):
                    w = one_vec(base + k * L, w)
                return w

            return lax.cond(has, hit, lambda w: w, w)

        ngrp = nvec // UNROLL
        n = lax.fori_loop(0, ngrp, group, jnp.int32(0))
        for k in range(nvec % UNROLL):
            n = one_vec((ngrp * UNROLL + k) * L, n)
        mlist_ref[pl.ds(n, L)] = jnp.zeros((L,), jnp.int32)
        return n

    n_a = scan_phase(dstA, mlA, CH_A // L)
    n_v = scan_phase(srcV, mlV, CH_V // L)

    def gather_phase(mlist_ref, n, pos_base, id_hbm, table_hbm, acc_row):
        nbat = (n + L - 1) // L

        def bt(b, _):
            gidx_v[...] = mlist_ref[pl.ds(b * L, L)] + pos_base
            pltpu.async_copy(id_hbm.at[gidx_v], sidx_v, sem).wait()
            pltpu.async_copy(table_hbm.at[sidx_v], rows, sem).wait()
            nval = jnp.minimum(n - b * L, L)

            def acc_it(j, _):
                for c in range(D // L):
                    pacc[acc_row, pl.ds(c * L, L)] += rows[j, pl.ds(c * L, L)]
                return 0

            lax.fori_loop(0, nval, acc_it, 0)
            return 0

        lax.fori_loop(0, nbat, bt, 0)

    gather_phase(mlA, n_a, wid * CH_A, aw_hbm, author_hbm, 0)
    gather_phase(mlV, n_v, wid * CH_V, pvd_hbm, venue_hbm, 1)

    cnt_v[0, pl.ds(0, L)] = jnp.full((L,), n_a, dtype=jnp.int32).astype(
        jnp.float32)
    cnt_v[1, pl.ds(0, L)] = jnp.full((L,), n_v, dtype=jnp.int32).astype(
        jnp.float32)

    pltpu.sync_copy(pacc, part_hbm.at[wid])
    pltpu.sync_copy(cnt_v, cnt_hbm.at[wid])

    plsc.subcore_barrier()

    pltpu.sync_copy(part_hbm, pslab)
    pltpu.sync_copy(cnt_hbm, clocal)

    ca = fzeros
    cv = fzeros
    sa = fzeros
    sv = fzeros
    for t in range(NW):
        ca = ca + clocal[t, 0, pl.ds(0, L)]
        cv = cv + clocal[t, 1, pl.ds(0, L)]
        sa = sa + pslab[t, 0, pl.ds(wid * L, L)]
        sv = sv + pslab[t, 1, pl.ds(wid * L, L)]

    one = jnp.ones((L,), jnp.float32)
    fa = jnp.where(ca > 0, wa_v[...] / jnp.maximum(ca, one), fzeros)
    fv = jnp.where(cv > 0, wv_v[...] / jnp.maximum(cv, one), fzeros)
    outv[...] = sa * fa + sv * fv
    pltpu.sync_copy(outv, out_hbm.at[pl.ds(wid * L, L)])


def _sc_call(pid16, wa16, wv16, aw, srcV, pvd, author_emb, venue_emb):
    mesh = plsc.VectorSubcoreMesh(core_axis_name="c", subcore_axis_name="s",
                                  num_cores=1, num_subcores=NW)
    cparams = pltpu.CompilerParams(needs_layout_passes=False)
    out, _, _ = pl.kernel(
        _fused_body,
        out_type=[
            jax.ShapeDtypeStruct((D,), jnp.float32),
            jax.ShapeDtypeStruct((NW, ACC_ROWS, D), jnp.float32),
            jax.ShapeDtypeStruct((NW, 2, L), jnp.float32),
        ],
        mesh=mesh,
        compiler_params=cparams,
        scratch_types=[
            pltpu.VMEM((L,), jnp.int32),
            pltpu.VMEM((CH_A,), jnp.int32),
            pltpu.VMEM((CH_V,), jnp.int32),
            pltpu.VMEM((CH_A + L,), jnp.int32),
            pltpu.VMEM((CH_V + L,), jnp.int32),
            pltpu.VMEM((L, D), jnp.float32),
            pltpu.VMEM((L,), jnp.int32),
            pltpu.VMEM((L,), jnp.int32),
            pltpu.VMEM((L,), jnp.int32),
            pltpu.VMEM((ACC_ROWS, D), jnp.float32),
            pltpu.VMEM((2, L), jnp.float32),
            pltpu.VMEM((L,), jnp.float32),
            pltpu.VMEM((L,), jnp.float32),
            pltpu.VMEM((NW, ACC_ROWS, D), jnp.float32),
            pltpu.VMEM((NW, 2, L), jnp.float32),
            pltpu.VMEM((L,), jnp.float32),
            pltpu.SemaphoreType.DMA,
        ],
    )(pid16, wa16, wv16, aw, srcV, pvd, author_emb, venue_emb)
    return out


def kernel(paper_id, year_idx, aw_edge_index, pv_src, pv_dst, paper_emb,
           author_emb, venue_emb, w_author, w_venue):
    del year_idx, paper_emb
    pid16 = jnp.full((L,), paper_id, dtype=jnp.int32)
    wa16 = jnp.full((L,), w_author, dtype=jnp.float32)
    wv16 = jnp.full((L,), w_venue, dtype=jnp.float32)
    aw = aw_edge_index.astype(jnp.int32).reshape(-1)
    return _sc_call(pid16, wa16, wv16, aw, pv_src.astype(jnp.int32),
                    pv_dst.astype(jnp.int32), author_emb, venue_emb)
